# Initial kernel scaffold; baseline (speedup 1.0000x reference)
import jax
import jax.numpy as jnp
from jax.experimental import pallas as pl


def kernel(x, coeffs, G_rows, G_cols, G_vals, EW, NS, L_rows, L_cols, L_vals, F_rows, F_cols, F_vals):
    raise NotImplementedError("write your pallas kernel here")


# speedup vs baseline: 58.1872x; 58.1872x over previous
import functools

import jax
import jax.numpy as jnp
from jax import lax
from jax.experimental import pallas as pl
from jax.experimental.pallas import tpu as pltpu
from jax.experimental.pallas import tpu_sc as plsc

NV = 40962
NF = 81920
B = 4
CIN = 32
COUT = 32
C128 = B * CIN

NW = 32
FPW = NF // NW
FCH = 8
NCH_F = FPW // FCH
VCH = 16
NCH_V = 81
NVP = NW * NCH_V * VCH
VPW = NCH_V * VCH


def _face_body(xt, gcols, gvals, ewf, nsf, fo,
               idx_v, gv_v, ew_v, ns_v, rows_v, obuf, sem):
  wid = lax.axis_index("s") * 2 + lax.axis_index("c")
  f_base = wid * FPW

  def chunk(t, carry):
    f0 = f_base + t * FCH
    pltpu.sync_copy(gcols.at[pl.ds(f0 * 9, FCH * 9)], idx_v)
    pltpu.sync_copy(gvals.at[pl.ds(f0 * 9, FCH * 9)], gv_v.at[pl.ds(0, FCH * 9)])
    pltpu.sync_copy(ewf.at[pl.ds(f0 * 3, FCH * 3)], ew_v.at[pl.ds(0, FCH * 3)])
    pltpu.sync_copy(nsf.at[pl.ds(f0 * 3, FCH * 3)], ns_v.at[pl.ds(0, FCH * 3)])
    pltpu.async_copy(xt.at[idx_v], rows_v, sem).wait()

    def face(fi, c2):
      wews = []
      wnss = []
      for k in range(9):
        gvb = plsc.load_gather(gv_v, [jnp.full((16,), fi * 9 + k, jnp.int32)])
        ewb = plsc.load_gather(ew_v, [jnp.full((16,), fi * 3 + k // 3, jnp.int32)])
        nsb = plsc.load_gather(ns_v, [jnp.full((16,), fi * 3 + k // 3, jnp.int32)])
        wews.append(gvb * ewb)
        wnss.append(gvb * nsb)
      for cg in range(8):
        r = rows_v[fi * 9, pl.ds(cg * 16, 16)]
        ae = wews[0] * r
        an = wnss[0] * r
        for k in range(1, 9):
          r = rows_v[fi * 9 + k, pl.ds(cg * 16, 16)]
          ae = ae + wews[k] * r
          an = an + wnss[k] * r
        obuf[fi, pl.ds(cg * 16, 16)] = ae
        obuf[fi, pl.ds(128 + cg * 16, 16)] = an
      return c2

    lax.fori_loop(0, FCH, face, 0)
    pltpu.sync_copy(obuf, fo.at[pl.ds(f0, FCH)])
    return carry

  lax.fori_loop(0, NCH_F, chunk, 0)


def _vert_body(xt, fo, lcols, lvals, fcols, fvals, res,
               lidx_v, lval_v, fidx_v, fval_v, lrows_v, frows_v, id_v, obuf,
               sem1, sem2):
  wid = lax.axis_index("s") * 2 + lax.axis_index("c")
  v_base = wid * VPW

  def chunk(t, carry):
    v0 = v_base + t * VCH
    pltpu.sync_copy(lcols.at[pl.ds(v0 * 7, VCH * 7)], lidx_v)
    pltpu.sync_copy(lvals.at[pl.ds(v0 * 7, VCH * 7)], lval_v.at[pl.ds(0, VCH * 7)])
    pltpu.sync_copy(fcols.at[pl.ds(v0 * 6, VCH * 6)], fidx_v)
    pltpu.sync_copy(fvals.at[pl.ds(v0 * 6, VCH * 6)], fval_v.at[pl.ds(0, VCH * 6)])
    pltpu.sync_copy(xt.at[pl.ds(v0, VCH)], id_v)
    cp1 = pltpu.async_copy(xt.at[lidx_v], lrows_v, sem1)
    cp2 = pltpu.async_copy(fo.at[fidx_v], frows_v, sem2)
    cp1.wait()
    cp2.wait()

    def vert(vi, c2):
      wls = [plsc.load_gather(lval_v, [jnp.full((16,), vi * 7 + k, jnp.int32)])
             for k in range(7)]
      wfs = [plsc.load_gather(fval_v, [jnp.full((16,), vi * 6 + k, jnp.int32)])
             for k in range(6)]
      for cg in range(8):
        b = cg // 2
        off = (cg % 2) * 16
        orow = vi * 4 + b
        obuf[orow, pl.ds(off, 16)] = id_v[vi, pl.ds(cg * 16, 16)]
        r = frows_v[vi * 6, pl.ds(cg * 16, 16)]
        ae = wfs[0] * r
        rn = frows_v[vi * 6, pl.ds(128 + cg * 16, 16)]
        an = wfs[0] * rn
        for k in range(1, 6):
          r = frows_v[vi * 6 + k, pl.ds(cg * 16, 16)]
          ae = ae + wfs[k] * r
          rn = frows_v[vi * 6 + k, pl.ds(128 + cg * 16, 16)]
          an = an + wfs[k] * rn
        obuf[orow, pl.ds(32 + off, 16)] = ae
        obuf[orow, pl.ds(64 + off, 16)] = an
        rl = lrows_v[vi * 7, pl.ds(cg * 16, 16)]
        al = wls[0] * rl
        for k in range(1, 7):
          rl = lrows_v[vi * 7 + k, pl.ds(cg * 16, 16)]
          al = al + wls[k] * rl
        obuf[orow, pl.ds(96 + off, 16)] = al
      return c2

    lax.fori_loop(0, VCH, vert, 0)
    pltpu.sync_copy(obuf, res.at[pl.ds(v0 * 4, VCH * 4)])
    return carry

  lax.fori_loop(0, NCH_V, chunk, 0)


def _mix_body(r_ref, w_ref, o_ref):
  o_ref[...] = jnp.dot(r_ref[...], w_ref[...],
                       preferred_element_type=jnp.float32)


@jax.jit
def kernel(x, coeffs, G_rows, G_cols, G_vals, EW, NS,
           L_rows, L_cols, L_vals, F_rows, F_cols, F_vals):
  del G_rows, L_rows, F_rows

  xt = x.reshape(C128, NV).T
  xt = jnp.pad(xt, ((0, NVP - NV), (0, 0)))

  gcols = G_cols.reshape(3, NF, 3).transpose(1, 0, 2).reshape(NF * 9)
  gvals = G_vals.reshape(3, NF, 3).transpose(1, 0, 2).reshape(NF * 9)
  ewf = EW.reshape(NF * 3)
  nsf = NS.reshape(NF * 3)

  lcols = jnp.pad(L_cols, (0, (NVP - NV) * 7))
  lvals = jnp.pad(L_vals, (0, (NVP - NV) * 7))
  fcols = jnp.pad(F_cols, (0, (NVP - NV) * 6))
  fvals = jnp.pad(F_vals, (0, (NVP - NV) * 6))

  mesh = plsc.VectorSubcoreMesh(core_axis_name="c", subcore_axis_name="s",
                                num_cores=2, num_subcores=16)

  sc_params = pltpu.CompilerParams(needs_layout_passes=False)

  fo = pl.kernel(
      _face_body,
      out_type=jax.ShapeDtypeStruct((NF, 256), jnp.float32),
      mesh=mesh,
      compiler_params=sc_params,
      scratch_types=[
          pltpu.VMEM((FCH * 9,), jnp.int32),
          pltpu.VMEM((128,), jnp.float32),
          pltpu.VMEM((128,), jnp.float32),
          pltpu.VMEM((128,), jnp.float32),
          pltpu.VMEM((FCH * 9, C128), jnp.float32),
          pltpu.VMEM((FCH, 256), jnp.float32),
          pltpu.SemaphoreType.DMA,
      ],
  )(xt, gcols, gvals, ewf, nsf)

  res = pl.kernel(
      _vert_body,
      out_type=jax.ShapeDtypeStruct((NVP * 4, C128), jnp.float32),
      mesh=mesh,
      compiler_params=sc_params,
      scratch_types=[
          pltpu.VMEM((VCH * 7,), jnp.int32),
          pltpu.VMEM((128,), jnp.float32),
          pltpu.VMEM((VCH * 6,), jnp.int32),
          pltpu.VMEM((128,), jnp.float32),
          pltpu.VMEM((VCH * 7, C128), jnp.float32),
          pltpu.VMEM((VCH * 6, 256), jnp.float32),
          pltpu.VMEM((VCH, C128), jnp.float32),
          pltpu.VMEM((VCH * 4, C128), jnp.float32),
          pltpu.SemaphoreType.DMA,
          pltpu.SemaphoreType.DMA,
      ],
  )(xt, fo, lcols, lvals, fcols, fvals)

  w = coeffs.transpose(2, 1, 0).reshape(4 * CIN, COUT)

  rows = NVP * 4
  blk = 2048
  out2 = pl.pallas_call(
      _mix_body,
      grid=(rows // blk,),
      in_specs=[
          pl.BlockSpec((blk, C128), lambda i: (i, 0)),
          pl.BlockSpec((4 * CIN, COUT), lambda i: (0, 0)),
      ],
      out_specs=pl.BlockSpec((blk, COUT), lambda i: (i, 0)),
      out_shape=jax.ShapeDtypeStruct((rows, COUT), jnp.float32),
  )(res, w)

  out = out2[: NV * 4].reshape(NV, B, COUT).transpose(1, 2, 0)
  return out


# speedup vs baseline: 87.3591x; 1.5013x over previous
import jax
import jax.numpy as jnp
from jax import lax
from jax.experimental import pallas as pl
from jax.experimental.pallas import tpu as pltpu
from jax.experimental.pallas import tpu_sc as plsc

NV = 40962
NF = 81920
B = 4
CIN = 32
COUT = 32
C128 = B * CIN

NW = 32
FCH = 8
NCH_F = 320
FPW = NCH_F * FCH
VCH = 16
NCH_V = 82
NVP = NW * NCH_V * VCH
VPW = NCH_V * VCH


def _face_compute(t, gv_v, ew_v, ns_v, rows_v, obuf):
  def face(fi, c2):
    tt = jnp.full((16,), t, jnp.int32)
    wews = []
    wnss = []
    for k in range(9):
      gvb = plsc.load_gather(gv_v, [tt, jnp.full((16,), fi * 9 + k, jnp.int32)])
      ewb = plsc.load_gather(ew_v, [tt, jnp.full((16,), fi * 3 + k // 3, jnp.int32)])
      nsb = plsc.load_gather(ns_v, [tt, jnp.full((16,), fi * 3 + k // 3, jnp.int32)])
      wews.append(gvb * ewb)
      wnss.append(gvb * nsb)
    for cg in range(8):
      r = rows_v[fi * 9, pl.ds(cg * 16, 16)]
      ae = wews[0] * r
      an = wnss[0] * r
      for k in range(1, 9):
        r = rows_v[fi * 9 + k, pl.ds(cg * 16, 16)]
        ae = ae + wews[k] * r
        an = an + wnss[k] * r
      obuf[fi, pl.ds(cg * 16, 16)] = ae
      obuf[fi, pl.ds(128 + cg * 16, 16)] = an
    return c2

  lax.fori_loop(0, FCH, face, 0)


def _face_body(xt, gcols, gvals, ewf, nsf, fo,
               gc_v, gv_v, ew_v, ns_v, rows_a, rows_b, obuf, sem_a, sem_b):
  wid = lax.axis_index("s") * 2 + lax.axis_index("c")
  f_base = wid * FPW
  c_base = wid * NCH_F

  pltpu.sync_copy(gcols.at[pl.ds(c_base, NCH_F)], gc_v)
  pltpu.sync_copy(gvals.at[pl.ds(c_base, NCH_F)], gv_v)
  pltpu.sync_copy(ewf.at[pl.ds(c_base, NCH_F)], ew_v)
  pltpu.sync_copy(nsf.at[pl.ds(c_base, NCH_F)], ns_v)

  pltpu.async_copy(xt.at[gc_v.at[0]], rows_a, sem_a)

  def pair(s, carry):
    t = 2 * s
    t2 = t + 1
    pltpu.async_copy(xt.at[gc_v.at[t2]], rows_b, sem_b)
    pltpu.make_async_copy(xt.at[gc_v.at[t]], rows_a, sem_a).wait()
    _face_compute(t, gv_v, ew_v, ns_v, rows_a, obuf)
    pltpu.sync_copy(obuf, fo.at[pl.ds(f_base + t * FCH, FCH)])

    @pl.when(s < NCH_F // 2 - 1)
    def _():
      pltpu.async_copy(xt.at[gc_v.at[t + 2]], rows_a, sem_a)

    pltpu.make_async_copy(xt.at[gc_v.at[t2]], rows_b, sem_b).wait()
    _face_compute(t2, gv_v, ew_v, ns_v, rows_b, obuf)
    pltpu.sync_copy(obuf, fo.at[pl.ds(f_base + t2 * FCH, FCH)])
    return carry

  lax.fori_loop(0, NCH_F // 2, pair, 0)


def _vert_compute(t, lv_v, fv_v, lrows_v, frows_v, id_v, obuf):
  def vert(vi, c2):
    tt = jnp.full((16,), t, jnp.int32)
    wls = [plsc.load_gather(lv_v, [tt, jnp.full((16,), vi * 7 + k, jnp.int32)])
           for k in range(7)]
    wfs = [plsc.load_gather(fv_v, [tt, jnp.full((16,), vi * 6 + k, jnp.int32)])
           for k in range(6)]
    for cg in range(8):
      b = cg // 2
      off = (cg % 2) * 16
      orow = vi * 4 + b
      obuf[orow, pl.ds(off, 16)] = id_v[vi, pl.ds(cg * 16, 16)]
      r = frows_v[vi * 6, pl.ds(cg * 16, 16)]
      ae = wfs[0] * r
      rn = frows_v[vi * 6, pl.ds(128 + cg * 16, 16)]
      an = wfs[0] * rn
      for k in range(1, 6):
        r = frows_v[vi * 6 + k, pl.ds(cg * 16, 16)]
        ae = ae + wfs[k] * r
        rn = frows_v[vi * 6 + k, pl.ds(128 + cg * 16, 16)]
        an = an + wfs[k] * rn
      obuf[orow, pl.ds(32 + off, 16)] = ae
      obuf[orow, pl.ds(64 + off, 16)] = an
      rl = lrows_v[vi * 7, pl.ds(cg * 16, 16)]
      al = wls[0] * rl
      for k in range(1, 7):
        rl = lrows_v[vi * 7 + k, pl.ds(cg * 16, 16)]
        al = al + wls[k] * rl
      obuf[orow, pl.ds(96 + off, 16)] = al
    return c2

  lax.fori_loop(0, VCH, vert, 0)


def _vert_body(xt, fo, lcols, lvals, fcols, fvals, res,
               lc_v, lv_v, fc_v, fv_v,
               lrows_a, lrows_b, frows_a, frows_b, id_a, id_b, obuf,
               sem_la, sem_lb, sem_fa, sem_fb, sem_ia, sem_ib):
  wid = lax.axis_index("s") * 2 + lax.axis_index("c")
  v_base = wid * VPW
  c_base = wid * NCH_V

  pltpu.sync_copy(lcols.at[pl.ds(c_base, NCH_V)], lc_v)
  pltpu.sync_copy(lvals.at[pl.ds(c_base, NCH_V)], lv_v)
  pltpu.sync_copy(fcols.at[pl.ds(c_base, NCH_V)], fc_v)
  pltpu.sync_copy(fvals.at[pl.ds(c_base, NCH_V)], fv_v)

  def issue(t, lrows_v, frows_v, id_v, sem_l, sem_f, sem_i):
    pltpu.async_copy(xt.at[lc_v.at[t]], lrows_v, sem_l)
    pltpu.async_copy(fo.at[fc_v.at[t]], frows_v, sem_f)
    pltpu.async_copy(xt.at[pl.ds(v_base + t * VCH, VCH)], id_v, sem_i)

  def wait(t, lrows_v, frows_v, id_v, sem_l, sem_f, sem_i):
    pltpu.make_async_copy(xt.at[lc_v.at[t]], lrows_v, sem_l).wait()
    pltpu.make_async_copy(fo.at[fc_v.at[t]], frows_v, sem_f).wait()
    pltpu.make_async_copy(xt.at[pl.ds(v_base, VCH)], id_v, sem_i).wait()

  issue(0, lrows_a, frows_a, id_a, sem_la, sem_fa, sem_ia)

  def pair(s, carry):
    t = 2 * s
    t2 = t + 1
    issue(t2, lrows_b, frows_b, id_b, sem_lb, sem_fb, sem_ib)
    wait(t, lrows_a, frows_a, id_a, sem_la, sem_fa, sem_ia)
    _vert_compute(t, lv_v, fv_v, lrows_a, frows_a, id_a, obuf)
    pltpu.sync_copy(obuf, res.at[pl.ds((v_base + t * VCH) * 4, VCH * 4)])

    @pl.when(s < NCH_V // 2 - 1)
    def _():
      issue(t + 2, lrows_a, frows_a, id_a, sem_la, sem_fa, sem_ia)

    wait(t2, lrows_b, frows_b, id_b, sem_lb, sem_fb, sem_ib)
    _vert_compute(t2, lv_v, fv_v, lrows_b, frows_b, id_b, obuf)
    pltpu.sync_copy(obuf, res.at[pl.ds((v_base + t2 * VCH) * 4, VCH * 4)])
    return carry

  lax.fori_loop(0, NCH_V // 2, pair, 0)


def _mix_body(r_ref, w_ref, o_ref):
  o_ref[...] = jnp.dot(r_ref[...], w_ref[...],
                       preferred_element_type=jnp.float32)


@jax.jit
def kernel(x, coeffs, G_rows, G_cols, G_vals, EW, NS,
           L_rows, L_cols, L_vals, F_rows, F_cols, F_vals):
  del G_rows, L_rows, F_rows

  xt = x.reshape(C128, NV).T
  xt = jnp.pad(xt, ((0, NVP - NV), (0, 0)))

  gcols = G_cols.reshape(3, NF, 3).transpose(1, 0, 2).reshape(NF // FCH, FCH * 9)
  gvals = G_vals.reshape(3, NF, 3).transpose(1, 0, 2).reshape(NF // FCH, FCH * 9)
  ewf = EW.reshape(NF // FCH, FCH * 3)
  nsf = NS.reshape(NF // FCH, FCH * 3)

  lcols = jnp.pad(L_cols, (0, (NVP - NV) * 7)).reshape(NVP // VCH, VCH * 7)
  lvals = jnp.pad(L_vals, (0, (NVP - NV) * 7)).reshape(NVP // VCH, VCH * 7)
  fcols = jnp.pad(F_cols, (0, (NVP - NV) * 6)).reshape(NVP // VCH, VCH * 6)
  fvals = jnp.pad(F_vals, (0, (NVP - NV) * 6)).reshape(NVP // VCH, VCH * 6)

  mesh = plsc.VectorSubcoreMesh(core_axis_name="c", subcore_axis_name="s",
                                num_cores=2, num_subcores=16)
  sc_params = pltpu.CompilerParams(needs_layout_passes=False,
                                   use_tc_tiling_on_sc=False)

  fo = pl.kernel(
      _face_body,
      out_type=jax.ShapeDtypeStruct((NF, 256), jnp.float32),
      mesh=mesh,
      compiler_params=sc_params,
      scratch_types=[
          pltpu.VMEM((NCH_F, FCH * 9), jnp.int32),
          pltpu.VMEM((NCH_F, FCH * 9), jnp.float32),
          pltpu.VMEM((NCH_F, FCH * 3), jnp.float32),
          pltpu.VMEM((NCH_F, FCH * 3), jnp.float32),
          pltpu.VMEM((FCH * 9, C128), jnp.float32),
          pltpu.VMEM((FCH * 9, C128), jnp.float32),
          pltpu.VMEM((FCH, 256), jnp.float32),
          pltpu.SemaphoreType.DMA,
          pltpu.SemaphoreType.DMA,
      ],
  )(xt, gcols, gvals, ewf, nsf)

  res = pl.kernel(
      _vert_body,
      out_type=jax.ShapeDtypeStruct((NVP * 4, C128), jnp.float32),
      mesh=mesh,
      compiler_params=sc_params,
      scratch_types=[
          pltpu.VMEM((NCH_V, VCH * 7), jnp.int32),
          pltpu.VMEM((NCH_V, VCH * 7), jnp.float32),
          pltpu.VMEM((NCH_V, VCH * 6), jnp.int32),
          pltpu.VMEM((NCH_V, VCH * 6), jnp.float32),
          pltpu.VMEM((VCH * 7, C128), jnp.float32),
          pltpu.VMEM((VCH * 7, C128), jnp.float32),
          pltpu.VMEM((VCH * 6, 256), jnp.float32),
          pltpu.VMEM((VCH * 6, 256), jnp.float32),
          pltpu.VMEM((VCH, C128), jnp.float32),
          pltpu.VMEM((VCH, C128), jnp.float32),
          pltpu.VMEM((VCH * 4, C128), jnp.float32),
          pltpu.SemaphoreType.DMA,
          pltpu.SemaphoreType.DMA,
          pltpu.SemaphoreType.DMA,
          pltpu.SemaphoreType.DMA,
          pltpu.SemaphoreType.DMA,
          pltpu.SemaphoreType.DMA,
      ],
  )(xt, fo, lcols, lvals, fcols, fvals)

  w = coeffs.transpose(2, 1, 0).reshape(4 * CIN, COUT)

  rows = NVP * 4
  blk = 2048
  out2 = pl.pallas_call(
      _mix_body,
      grid=(rows // blk,),
      in_specs=[
          pl.BlockSpec((blk, C128), lambda i: (i, 0)),
          pl.BlockSpec((4 * CIN, COUT), lambda i: (0, 0)),
      ],
      out_specs=pl.BlockSpec((blk, COUT), lambda i: (i, 0)),
      out_shape=jax.ShapeDtypeStruct((rows, COUT), jnp.float32),
  )(res, w)

  out = out2[: NV * 4].reshape(NV, B, COUT).transpose(1, 2, 0)
  return out


# speedup vs baseline: 136.1015x; 1.5580x over previous
import jax
import jax.numpy as jnp
from jax import lax
from jax.experimental import pallas as pl
from jax.experimental.pallas import tpu as pltpu
from jax.experimental.pallas import tpu_sc as plsc

NV = 40962
NF = 81920
B = 4
CIN = 32
COUT = 32
C128 = B * CIN

NW = 32
FCH = 8
NCH_F = 320
FPW = NCH_F * FCH
VCH = 16
NCH_V = 82
NVP = NW * NCH_V * VCH
VPW = NCH_V * VCH


def _tr_body(x_ref, o_ref):
  o_ref[...] = x_ref[...].T


def _face_compute(t, gv0, gv1, gv2, ew_v, ns_v, rows_v, obuf):
  gvs = (gv0, gv1, gv2)

  def face(fi, c2):
    tt = jnp.full((16,), t, jnp.int32)
    wews = []
    wnss = []
    for k in range(9):
      j, u = k // 3, k % 3
      gvb = plsc.load_gather(gvs[j], [tt, jnp.full((16,), fi * 3 + u, jnp.int32)])
      ewb = plsc.load_gather(ew_v, [tt, jnp.full((16,), fi * 3 + j, jnp.int32)])
      nsb = plsc.load_gather(ns_v, [tt, jnp.full((16,), fi * 3 + j, jnp.int32)])
      wews.append(gvb * ewb)
      wnss.append(gvb * nsb)
    for cg in range(8):
      r = rows_v[fi * 3, pl.ds(cg * 16, 16)]
      ae = wews[0] * r
      an = wnss[0] * r
      for k in range(1, 9):
        j, u = k // 3, k % 3
        r = rows_v[j * 24 + fi * 3 + u, pl.ds(cg * 16, 16)]
        ae = ae + wews[k] * r
        an = an + wnss[k] * r
      obuf[fi, pl.ds(cg * 16, 16)] = ae
      obuf[fi, pl.ds(128 + cg * 16, 16)] = an
    return c2

  lax.fori_loop(0, FCH, face, 0)


def _face_body(xt, gcols, gvals, ewf, nsf, fo,
               gc0, gc1, gc2, gv0, gv1, gv2, ew_v, ns_v,
               rows_a, rows_b, obuf, sem_a, sem_b):
  wid = lax.axis_index("s") * 2 + lax.axis_index("c")
  f_base = wid * FPW
  c_base = wid * NCH_F

  for j, (gc_v, gv_v) in enumerate(((gc0, gv0), (gc1, gv1), (gc2, gv2))):
    pltpu.sync_copy(gcols.at[j, pl.ds(c_base, NCH_F)], gc_v)
    pltpu.sync_copy(gvals.at[j, pl.ds(c_base, NCH_F)], gv_v)
  pltpu.sync_copy(ewf.at[pl.ds(c_base, NCH_F)], ew_v)
  pltpu.sync_copy(nsf.at[pl.ds(c_base, NCH_F)], ns_v)

  gcs = (gc0, gc1, gc2)

  def issue(t, rows_v, sem):
    for j in range(3):
      pltpu.async_copy(xt.at[gcs[j].at[t]], rows_v.at[pl.ds(j * 24, 24)], sem)

  def wait(t, rows_v, sem):
    for j in range(3):
      pltpu.make_async_copy(
          xt.at[gcs[j].at[t]], rows_v.at[pl.ds(j * 24, 24)], sem).wait()

  issue(0, rows_a, sem_a)

  def pair(s, carry):
    t = 2 * s
    t2 = t + 1
    issue(t2, rows_b, sem_b)
    wait(t, rows_a, sem_a)
    _face_compute(t, gv0, gv1, gv2, ew_v, ns_v, rows_a, obuf)
    pltpu.sync_copy(obuf, fo.at[pl.ds(f_base + t * FCH, FCH)])

    @pl.when(s < NCH_F // 2 - 1)
    def _():
      issue(t + 2, rows_a, sem_a)

    wait(t2, rows_b, sem_b)
    _face_compute(t2, gv0, gv1, gv2, ew_v, ns_v, rows_b, obuf)
    pltpu.sync_copy(obuf, fo.at[pl.ds(f_base + t2 * FCH, FCH)])
    return carry

  lax.fori_loop(0, NCH_F // 2, pair, 0)


def _vert_compute(t, lv_v, fv_v, lrows_v, frows_v, id_v, obuf):
  def vert(vi, c2):
    tt = jnp.full((16,), t, jnp.int32)
    wls = [plsc.load_gather(lv_v, [tt, jnp.full((16,), vi * 7 + k, jnp.int32)])
           for k in range(7)]
    wfs = [plsc.load_gather(fv_v, [tt, jnp.full((16,), vi * 6 + k, jnp.int32)])
           for k in range(6)]
    for cg in range(8):
      b = cg // 2
      off = (cg % 2) * 16
      orow = b * VCH + vi
      obuf[orow, pl.ds(off, 16)] = id_v[vi, pl.ds(cg * 16, 16)]
      r = frows_v[vi * 6, pl.ds(cg * 16, 16)]
      ae = wfs[0] * r
      rn = frows_v[vi * 6, pl.ds(128 + cg * 16, 16)]
      an = wfs[0] * rn
      for k in range(1, 6):
        r = frows_v[vi * 6 + k, pl.ds(cg * 16, 16)]
        ae = ae + wfs[k] * r
        rn = frows_v[vi * 6 + k, pl.ds(128 + cg * 16, 16)]
        an = an + wfs[k] * rn
      obuf[orow, pl.ds(32 + off, 16)] = ae
      obuf[orow, pl.ds(64 + off, 16)] = an
      rl = lrows_v[vi * 7, pl.ds(cg * 16, 16)]
      al = wls[0] * rl
      for k in range(1, 7):
        rl = lrows_v[vi * 7 + k, pl.ds(cg * 16, 16)]
        al = al + wls[k] * rl
      obuf[orow, pl.ds(96 + off, 16)] = al
    return c2

  lax.fori_loop(0, VCH, vert, 0)


def _vert_body(xt, fo, lcols, lvals, fcols, fvals, res,
               lc_v, lv_v, fc_v, fv_v,
               lrows_a, lrows_b, frows_a, frows_b, id_a, id_b, obuf,
               sem_la, sem_lb, sem_fa, sem_fb, sem_ia, sem_ib):
  wid = lax.axis_index("s") * 2 + lax.axis_index("c")
  v_base = wid * VPW
  c_base = wid * NCH_V

  pltpu.sync_copy(lcols.at[pl.ds(c_base, NCH_V)], lc_v)
  pltpu.sync_copy(lvals.at[pl.ds(c_base, NCH_V)], lv_v)
  pltpu.sync_copy(fcols.at[pl.ds(c_base, NCH_V)], fc_v)
  pltpu.sync_copy(fvals.at[pl.ds(c_base, NCH_V)], fv_v)

  def issue(t, lrows_v, frows_v, id_v, sem_l, sem_f, sem_i):
    pltpu.async_copy(xt.at[lc_v.at[t]], lrows_v, sem_l)
    pltpu.async_copy(fo.at[fc_v.at[t]], frows_v, sem_f)
    pltpu.async_copy(xt.at[pl.ds(v_base + t * VCH, VCH)], id_v, sem_i)

  def wait(t, lrows_v, frows_v, id_v, sem_l, sem_f, sem_i):
    pltpu.make_async_copy(xt.at[lc_v.at[t]], lrows_v, sem_l).wait()
    pltpu.make_async_copy(fo.at[fc_v.at[t]], frows_v, sem_f).wait()
    pltpu.make_async_copy(xt.at[pl.ds(v_base, VCH)], id_v, sem_i).wait()

  issue(0, lrows_a, frows_a, id_a, sem_la, sem_fa, sem_ia)

  def out_write(t, obuf):
    for b in range(B):
      pltpu.sync_copy(obuf.at[pl.ds(b * VCH, VCH)],
                      res.at[b, pl.ds(v_base + t * VCH, VCH)])

  def pair(s, carry):
    t = 2 * s
    t2 = t + 1
    issue(t2, lrows_b, frows_b, id_b, sem_lb, sem_fb, sem_ib)
    wait(t, lrows_a, frows_a, id_a, sem_la, sem_fa, sem_ia)
    _vert_compute(t, lv_v, fv_v, lrows_a, frows_a, id_a, obuf)
    out_write(t, obuf)

    @pl.when(s < NCH_V // 2 - 1)
    def _():
      issue(t + 2, lrows_a, frows_a, id_a, sem_la, sem_fa, sem_ia)

    wait(t2, lrows_b, frows_b, id_b, sem_lb, sem_fb, sem_ib)
    _vert_compute(t2, lv_v, fv_v, lrows_b, frows_b, id_b, obuf)
    out_write(t2, obuf)
    return carry

  lax.fori_loop(0, NCH_V // 2, pair, 0)


def _mix_body(wt_ref, r_ref, o_ref):
  o_ref[0] = jax.lax.dot_general(
      wt_ref[...], r_ref[0],
      dimension_numbers=(((1,), (1,)), ((), ())),
      preferred_element_type=jnp.float32)


@jax.jit
def kernel(x, coeffs, G_rows, G_cols, G_vals, EW, NS,
           L_rows, L_cols, L_vals, F_rows, F_cols, F_vals):
  del G_rows, L_rows, F_rows

  x2 = x.reshape(C128, NV)
  xt = pl.pallas_call(
      _tr_body,
      grid=((NV + 511) // 512,),
      in_specs=[pl.BlockSpec((C128, 512), lambda i: (0, i))],
      out_specs=pl.BlockSpec((512, C128), lambda i: (i, 0)),
      out_shape=jax.ShapeDtypeStruct((NVP, C128), jnp.float32),
  )(x2)

  gcols = G_cols.reshape(3, NF // FCH, FCH * 3)
  gvals = G_vals.reshape(3, NF // FCH, FCH * 3)
  ewf = EW.reshape(NF // FCH, FCH * 3)
  nsf = NS.reshape(NF // FCH, FCH * 3)

  lcols = jnp.pad(L_cols, (0, (NVP - NV) * 7)).reshape(NVP // VCH, VCH * 7)
  lvals = jnp.pad(L_vals, (0, (NVP - NV) * 7)).reshape(NVP // VCH, VCH * 7)
  fcols = jnp.pad(F_cols, (0, (NVP - NV) * 6)).reshape(NVP // VCH, VCH * 6)
  fvals = jnp.pad(F_vals, (0, (NVP - NV) * 6)).reshape(NVP // VCH, VCH * 6)

  mesh = plsc.VectorSubcoreMesh(core_axis_name="c", subcore_axis_name="s",
                                num_cores=2, num_subcores=16)
  sc_params = pltpu.CompilerParams(needs_layout_passes=False,
                                   use_tc_tiling_on_sc=False)

  fo = pl.kernel(
      _face_body,
      out_type=jax.ShapeDtypeStruct((NF, 256), jnp.float32),
      mesh=mesh,
      compiler_params=sc_params,
      scratch_types=[
          pltpu.VMEM((NCH_F, FCH * 3), jnp.int32),
          pltpu.VMEM((NCH_F, FCH * 3), jnp.int32),
          pltpu.VMEM((NCH_F, FCH * 3), jnp.int32),
          pltpu.VMEM((NCH_F, FCH * 3), jnp.float32),
          pltpu.VMEM((NCH_F, FCH * 3), jnp.float32),
          pltpu.VMEM((NCH_F, FCH * 3), jnp.float32),
          pltpu.VMEM((NCH_F, FCH * 3), jnp.float32),
          pltpu.VMEM((NCH_F, FCH * 3), jnp.float32),
          pltpu.VMEM((FCH * 9, C128), jnp.float32),
          pltpu.VMEM((FCH * 9, C128), jnp.float32),
          pltpu.VMEM((FCH, 256), jnp.float32),
          pltpu.SemaphoreType.DMA,
          pltpu.SemaphoreType.DMA,
      ],
  )(xt, gcols, gvals, ewf, nsf)

  res = pl.kernel(
      _vert_body,
      out_type=jax.ShapeDtypeStruct((B, NVP, C128), jnp.float32),
      mesh=mesh,
      compiler_params=sc_params,
      scratch_types=[
          pltpu.VMEM((NCH_V, VCH * 7), jnp.int32),
          pltpu.VMEM((NCH_V, VCH * 7), jnp.float32),
          pltpu.VMEM((NCH_V, VCH * 6), jnp.int32),
          pltpu.VMEM((NCH_V, VCH * 6), jnp.float32),
          pltpu.VMEM((VCH * 7, C128), jnp.float32),
          pltpu.VMEM((VCH * 7, C128), jnp.float32),
          pltpu.VMEM((VCH * 6, 256), jnp.float32),
          pltpu.VMEM((VCH * 6, 256), jnp.float32),
          pltpu.VMEM((VCH, C128), jnp.float32),
          pltpu.VMEM((VCH, C128), jnp.float32),
          pltpu.VMEM((B * VCH, C128), jnp.float32),
          pltpu.SemaphoreType.DMA,
          pltpu.SemaphoreType.DMA,
          pltpu.SemaphoreType.DMA,
          pltpu.SemaphoreType.DMA,
          pltpu.SemaphoreType.DMA,
          pltpu.SemaphoreType.DMA,
      ],
  )(xt, fo, lcols, lvals, fcols, fvals)

  w_t = coeffs.transpose(0, 2, 1).reshape(COUT, 4 * CIN)

  nblk = (NV + 511) // 512
  out = pl.pallas_call(
      _mix_body,
      grid=(B, nblk),
      in_specs=[
          pl.BlockSpec((COUT, 4 * CIN), lambda b, i: (0, 0)),
          pl.BlockSpec((1, 512, C128), lambda b, i: (b, i, 0)),
      ],
      out_specs=pl.BlockSpec((1, COUT, 512), lambda b, i: (b, 0, i)),
      out_shape=jax.ShapeDtypeStruct((B, COUT, NV), jnp.float32),
  )(w_t, res)

  return out


# speedup vs baseline: 150.9501x; 1.1091x over previous
import jax
import jax.numpy as jnp
from jax import lax
from jax.experimental import pallas as pl
from jax.experimental.pallas import tpu as pltpu
from jax.experimental.pallas import tpu_sc as plsc

NV = 40962
NF = 81920
B = 4
CIN = 32
COUT = 32
C128 = B * CIN

NW = 32
FCH = 8
NCH_F = 320
FPW = NCH_F * FCH
VCH = 16
NCH_V = 82
NVP = NW * NCH_V * VCH
VPW = NCH_V * VCH


def _tr_body(x_ref, o_ref):
  o_ref[...] = x_ref[...].T


def _face_compute(t, gv0, gv1, gv2, ew_v, ns_v, rows_v, obuf_e, obuf_n):
  gvs = (gv0, gv1, gv2)

  def face(fi, c2):
    gw = []
    ewj = []
    nsj = []
    for j in range(3):
      for u in range(3):
        gw.append(plsc.load_gather(
            gvs[j], [jnp.full((16,), t * 24 + fi * 3 + u, jnp.int32)]))
      ewj.append(plsc.load_gather(
          ew_v, [jnp.full((16,), t * 24 + fi * 3 + j, jnp.int32)]))
      nsj.append(plsc.load_gather(
          ns_v, [jnp.full((16,), t * 24 + fi * 3 + j, jnp.int32)]))
    for cg in range(8):
      gj = []
      for j in range(3):
        r = rows_v[j * 24 + fi * 3, pl.ds(cg * 16, 16)]
        g = gw[j * 3] * r
        for u in range(1, 3):
          r = rows_v[j * 24 + fi * 3 + u, pl.ds(cg * 16, 16)]
          g = g + gw[j * 3 + u] * r
        gj.append(g)
      ae = ewj[0] * gj[0] + ewj[1] * gj[1] + ewj[2] * gj[2]
      an = nsj[0] * gj[0] + nsj[1] * gj[1] + nsj[2] * gj[2]
      obuf_e[fi, pl.ds(cg * 16, 16)] = ae
      obuf_n[fi, pl.ds(cg * 16, 16)] = an
    return c2

  lax.fori_loop(0, FCH, face, 0)


def _face_body(xt, gcols, gvals, ewf, nsf, fo_e, fo_n,
               gc0, gc1, gc2, gv0, gv1, gv2, ew_v, ns_v,
               rows_a, rows_b, obuf_e, obuf_n, sem_a, sem_b):
  wid = lax.axis_index("s") * 2 + lax.axis_index("c")
  f_base = wid * FPW

  for j, (gc_v, gv_v) in enumerate(((gc0, gv0), (gc1, gv1), (gc2, gv2))):
    pltpu.sync_copy(gcols.at[pl.ds(j * NF * 3 + f_base * 3, FPW * 3)], gc_v)
    pltpu.sync_copy(gvals.at[pl.ds(j * NF * 3 + f_base * 3, FPW * 3)], gv_v)
  pltpu.sync_copy(ewf.at[pl.ds(f_base * 3, FPW * 3)], ew_v)
  pltpu.sync_copy(nsf.at[pl.ds(f_base * 3, FPW * 3)], ns_v)

  gcs = (gc0, gc1, gc2)

  def issue(t, rows_v, sem):
    for j in range(3):
      pltpu.async_copy(xt.at[gcs[j].at[pl.ds(t * 24, 24)]],
                       rows_v.at[pl.ds(j * 24, 24)], sem)

  def wait(t, rows_v, sem):
    for j in range(3):
      pltpu.make_async_copy(
          xt.at[gcs[j].at[pl.ds(t * 24, 24)]],
          rows_v.at[pl.ds(j * 24, 24)], sem).wait()

  issue(0, rows_a, sem_a)

  def pair(s, carry):
    t = 2 * s
    t2 = t + 1
    issue(t2, rows_b, sem_b)
    wait(t, rows_a, sem_a)
    _face_compute(t, gv0, gv1, gv2, ew_v, ns_v, rows_a, obuf_e, obuf_n)
    pltpu.sync_copy(obuf_e, fo_e.at[pl.ds(f_base + t * FCH, FCH)])
    pltpu.sync_copy(obuf_n, fo_n.at[pl.ds(f_base + t * FCH, FCH)])

    @pl.when(s < NCH_F // 2 - 1)
    def _():
      issue(t + 2, rows_a, sem_a)

    wait(t2, rows_b, sem_b)
    _face_compute(t2, gv0, gv1, gv2, ew_v, ns_v, rows_b, obuf_e, obuf_n)
    pltpu.sync_copy(obuf_e, fo_e.at[pl.ds(f_base + t2 * FCH, FCH)])
    pltpu.sync_copy(obuf_n, fo_n.at[pl.ds(f_base + t2 * FCH, FCH)])
    return carry

  lax.fori_loop(0, NCH_F // 2, pair, 0)


def _vert_compute(t, lv_v, fv_v, lrows_v, frows_e, frows_n, id_v, obuf):
  def vert(vi, c2):
    wls = [plsc.load_gather(
        lv_v, [jnp.full((16,), t * 112 + vi * 7 + k, jnp.int32)])
        for k in range(7)]
    wfs = [plsc.load_gather(
        fv_v, [jnp.full((16,), t * 96 + vi * 6 + k, jnp.int32)])
        for k in range(6)]
    for cg in range(8):
      b = cg // 2
      off = (cg % 2) * 16
      orow = b * VCH + vi
      obuf[orow, pl.ds(off, 16)] = id_v[vi, pl.ds(cg * 16, 16)]
      r = frows_e[vi * 6, pl.ds(cg * 16, 16)]
      ae = wfs[0] * r
      rn = frows_n[vi * 6, pl.ds(cg * 16, 16)]
      an = wfs[0] * rn
      for k in range(1, 6):
        r = frows_e[vi * 6 + k, pl.ds(cg * 16, 16)]
        ae = ae + wfs[k] * r
        rn = frows_n[vi * 6 + k, pl.ds(cg * 16, 16)]
        an = an + wfs[k] * rn
      obuf[orow, pl.ds(32 + off, 16)] = ae
      obuf[orow, pl.ds(64 + off, 16)] = an
      rl = lrows_v[vi * 7, pl.ds(cg * 16, 16)]
      al = wls[0] * rl
      for k in range(1, 7):
        rl = lrows_v[vi * 7 + k, pl.ds(cg * 16, 16)]
        al = al + wls[k] * rl
      obuf[orow, pl.ds(96 + off, 16)] = al
    return c2

  lax.fori_loop(0, VCH, vert, 0)


def _vert_body(xt, fo_e, fo_n, lcols, lvals, fcols, fvals, res,
               lc_v, lv_v, fc_v, fv_v,
               lrows_a, lrows_b, fre_a, fre_b, frn_a, frn_b, id_a, id_b, obuf,
               sem_la, sem_lb, sem_fa, sem_fb, sem_ia, sem_ib):
  wid = lax.axis_index("s") * 2 + lax.axis_index("c")
  v_base = wid * VPW

  pltpu.sync_copy(lcols.at[pl.ds(v_base * 7, VPW * 7)], lc_v)
  pltpu.sync_copy(lvals.at[pl.ds(v_base * 7, VPW * 7)], lv_v)
  pltpu.sync_copy(fcols.at[pl.ds(v_base * 6, VPW * 6)], fc_v)
  pltpu.sync_copy(fvals.at[pl.ds(v_base * 6, VPW * 6)], fv_v)

  def issue(t, lrows_v, fre, frn, id_v, sem_l, sem_f, sem_i):
    pltpu.async_copy(xt.at[lc_v.at[pl.ds(t * 112, 112)]], lrows_v, sem_l)
    pltpu.async_copy(fo_e.at[fc_v.at[pl.ds(t * 96, 96)]], fre, sem_f)
    pltpu.async_copy(fo_n.at[fc_v.at[pl.ds(t * 96, 96)]], frn, sem_f)
    pltpu.async_copy(xt.at[pl.ds(v_base + t * VCH, VCH)], id_v, sem_i)

  def wait(t, lrows_v, fre, frn, id_v, sem_l, sem_f, sem_i):
    pltpu.make_async_copy(
        xt.at[lc_v.at[pl.ds(t * 112, 112)]], lrows_v, sem_l).wait()
    pltpu.make_async_copy(
        fo_e.at[fc_v.at[pl.ds(t * 96, 96)]], fre, sem_f).wait()
    pltpu.make_async_copy(
        fo_n.at[fc_v.at[pl.ds(t * 96, 96)]], frn, sem_f).wait()
    pltpu.make_async_copy(xt.at[pl.ds(v_base, VCH)], id_v, sem_i).wait()

  issue(0, lrows_a, fre_a, frn_a, id_a, sem_la, sem_fa, sem_ia)

  def out_write(t, obuf):
    for b in range(B):
      pltpu.sync_copy(obuf.at[pl.ds(b * VCH, VCH)],
                      res.at[b, pl.ds(v_base + t * VCH, VCH)])

  def pair(s, carry):
    t = 2 * s
    t2 = t + 1
    issue(t2, lrows_b, fre_b, frn_b, id_b, sem_lb, sem_fb, sem_ib)
    wait(t, lrows_a, fre_a, frn_a, id_a, sem_la, sem_fa, sem_ia)
    _vert_compute(t, lv_v, fv_v, lrows_a, fre_a, frn_a, id_a, obuf)
    out_write(t, obuf)

    @pl.when(s < NCH_V // 2 - 1)
    def _():
      issue(t + 2, lrows_a, fre_a, frn_a, id_a, sem_la, sem_fa, sem_ia)

    wait(t2, lrows_b, fre_b, frn_b, id_b, sem_lb, sem_fb, sem_ib)
    _vert_compute(t2, lv_v, fv_v, lrows_b, fre_b, frn_b, id_b, obuf)
    out_write(t2, obuf)
    return carry

  lax.fori_loop(0, NCH_V // 2, pair, 0)


def _mix_body(wt_ref, r_ref, o_ref):
  o_ref[0] = jax.lax.dot_general(
      wt_ref[...], r_ref[0],
      dimension_numbers=(((1,), (1,)), ((), ())),
      preferred_element_type=jnp.float32)


@jax.jit
def kernel(x, coeffs, G_rows, G_cols, G_vals, EW, NS,
           L_rows, L_cols, L_vals, F_rows, F_cols, F_vals):
  del G_rows, L_rows, F_rows

  x2 = x.reshape(C128, NV)
  xt = pl.pallas_call(
      _tr_body,
      grid=((NV + 511) // 512,),
      in_specs=[pl.BlockSpec((C128, 512), lambda i: (0, i))],
      out_specs=pl.BlockSpec((512, C128), lambda i: (i, 0)),
      out_shape=jax.ShapeDtypeStruct((NVP, C128), jnp.float32),
  )(x2)

  ewf = EW.reshape(NF * 3)
  nsf = NS.reshape(NF * 3)

  lcols = jnp.pad(L_cols, (0, (NVP - NV) * 7))
  lvals = jnp.pad(L_vals, (0, (NVP - NV) * 7))
  fcols = jnp.pad(F_cols, (0, (NVP - NV) * 6))
  fvals = jnp.pad(F_vals, (0, (NVP - NV) * 6))

  mesh = plsc.VectorSubcoreMesh(core_axis_name="c", subcore_axis_name="s",
                                num_cores=2, num_subcores=16)
  sc_params = pltpu.CompilerParams(needs_layout_passes=False,
                                   use_tc_tiling_on_sc=False)

  fo_e, fo_n = pl.kernel(
      _face_body,
      out_type=(jax.ShapeDtypeStruct((NF, C128), jnp.float32),
                jax.ShapeDtypeStruct((NF, C128), jnp.float32)),
      mesh=mesh,
      compiler_params=sc_params,
      scratch_types=[
          pltpu.VMEM((FPW * 3,), jnp.int32),
          pltpu.VMEM((FPW * 3,), jnp.int32),
          pltpu.VMEM((FPW * 3,), jnp.int32),
          pltpu.VMEM((FPW * 3,), jnp.float32),
          pltpu.VMEM((FPW * 3,), jnp.float32),
          pltpu.VMEM((FPW * 3,), jnp.float32),
          pltpu.VMEM((FPW * 3,), jnp.float32),
          pltpu.VMEM((FPW * 3,), jnp.float32),
          pltpu.VMEM((FCH * 9, C128), jnp.float32),
          pltpu.VMEM((FCH * 9, C128), jnp.float32),
          pltpu.VMEM((FCH, C128), jnp.float32),
          pltpu.VMEM((FCH, C128), jnp.float32),
          pltpu.SemaphoreType.DMA,
          pltpu.SemaphoreType.DMA,
      ],
  )(xt, G_cols, G_vals, ewf, nsf)

  res = pl.kernel(
      _vert_body,
      out_type=jax.ShapeDtypeStruct((B, NVP, C128), jnp.float32),
      mesh=mesh,
      compiler_params=sc_params,
      scratch_types=[
          pltpu.VMEM((VPW * 7,), jnp.int32),
          pltpu.VMEM((VPW * 7,), jnp.float32),
          pltpu.VMEM((VPW * 6,), jnp.int32),
          pltpu.VMEM((VPW * 6,), jnp.float32),
          pltpu.VMEM((VCH * 7, C128), jnp.float32),
          pltpu.VMEM((VCH * 7, C128), jnp.float32),
          pltpu.VMEM((VCH * 6, C128), jnp.float32),
          pltpu.VMEM((VCH * 6, C128), jnp.float32),
          pltpu.VMEM((VCH * 6, C128), jnp.float32),
          pltpu.VMEM((VCH * 6, C128), jnp.float32),
          pltpu.VMEM((VCH, C128), jnp.float32),
          pltpu.VMEM((VCH, C128), jnp.float32),
          pltpu.VMEM((B * VCH, C128), jnp.float32),
          pltpu.SemaphoreType.DMA,
          pltpu.SemaphoreType.DMA,
          pltpu.SemaphoreType.DMA,
          pltpu.SemaphoreType.DMA,
          pltpu.SemaphoreType.DMA,
          pltpu.SemaphoreType.DMA,
      ],
  )(xt, fo_e, fo_n, lcols, lvals, fcols, fvals)

  w_t = coeffs.transpose(0, 2, 1).reshape(COUT, 4 * CIN)

  nblk = (NV + 2047) // 2048
  out = pl.pallas_call(
      _mix_body,
      grid=(B, nblk),
      in_specs=[
          pl.BlockSpec((COUT, 4 * CIN), lambda b, i: (0, 0)),
          pl.BlockSpec((1, 2048, C128), lambda b, i: (b, i, 0)),
      ],
      out_specs=pl.BlockSpec((1, COUT, 2048), lambda b, i: (b, 0, i)),
      out_shape=jax.ShapeDtypeStruct((B, COUT, NV), jnp.float32),
  )(w_t, res)

  return out
